# Initial kernel scaffold; baseline (speedup 1.0000x reference)
import jax
import jax.numpy as jnp
from jax.experimental import pallas as pl


def kernel(x, edge_index, W1, b1, W2, b2, Wfc, bfc):
    raise NotImplementedError("write your pallas kernel here")


# speedup vs baseline: 35.4016x; 35.4016x over previous
import functools

import jax
import jax.numpy as jnp
from jax import lax
from jax.experimental import pallas as pl
from jax.experimental.pallas import tpu as pltpu
from jax.experimental.pallas import tpu_sc as plsc

N_NODES = 10000
N_EDGES = 320000
NUM_CORES = 2
NUM_SUBCORES = 16
NW = NUM_CORES * NUM_SUBCORES
EPT = 10240
ROWS = 128
NJ = EPT // ROWS
NACC = 10240
ZROWS = NACC // NUM_SUBCORES
TRASH = 10100
C = 8


def _sc_mesh():
  return plsc.VectorSubcoreMesh(core_axis_name="c", subcore_axis_name="s",
                                num_cores=NUM_CORES, num_subcores=NUM_SUBCORES)


_SC_PARAMS = pltpu.CompilerParams(use_tc_tiling_on_sc=False)


def _scatter_body(src_h, dst_h, table_h, zeros_h, out_h,
                  srcv, dstv, msg, buf, acc, semg, sems):
  c = lax.axis_index("c")
  s = lax.axis_index("s")
  wid = c * NUM_SUBCORES + s

  pltpu.sync_copy(src_h.at[wid], srcv)
  pltpu.sync_copy(dst_h.at[wid], dstv)

  pltpu.sync_copy(zeros_h, buf)
  pltpu.sync_copy(buf, acc.at[pl.ds(s * ZROWS, ZROWS)])
  plsc.subcore_barrier()

  for j in range(NJ):
    pltpu.async_copy(table_h.at[srcv.at[j]], msg.at[j], semg)
  for j in range(NJ):
    pltpu.make_async_copy(table_h.at[srcv.at[j]], msg.at[j], semg).wait()

  for j in range(NJ):
    pltpu.async_copy(msg.at[j], acc.at[dstv.at[j]], sems, add=True)
  for j in range(NJ):
    pltpu.make_async_copy(msg.at[j], acc.at[dstv.at[j]], sems).wait()
  plsc.subcore_barrier()

  pltpu.sync_copy(acc.at[pl.ds(s * ZROWS, ZROWS)], buf)
  pltpu.sync_copy(buf, out_h.at[c, pl.ds(s * ZROWS, ZROWS)])


def _make_scatter():
  return pl.kernel(
      _scatter_body,
      out_type=jax.ShapeDtypeStruct((NUM_CORES, NACC, C), jnp.float32),
      mesh=_sc_mesh(),
      scratch_types=[
          pltpu.VMEM((NJ, ROWS), jnp.int32),
          pltpu.VMEM((NJ, ROWS), jnp.int32),
          pltpu.VMEM((NJ, ROWS, C), jnp.float32),
          pltpu.VMEM((ZROWS, C), jnp.float32),
          pltpu.VMEM_SHARED((NACC, C), jnp.float32),
          pltpu.SemaphoreType.DMA,
          pltpu.SemaphoreType.DMA,
      ],
      compiler_params=_SC_PARAMS,
      name="gcn_scatter",
  )


def _deg_body(dst_h, ones_h, zeros_h, out_h, dstv, onesv, buf, acc, sems):
  c = lax.axis_index("c")
  s = lax.axis_index("s")
  wid = c * NUM_SUBCORES + s

  pltpu.sync_copy(dst_h.at[wid], dstv)
  pltpu.sync_copy(ones_h, onesv)
  pltpu.sync_copy(zeros_h, buf)
  pltpu.sync_copy(buf, acc.at[pl.ds(s * ZROWS, ZROWS)])
  plsc.subcore_barrier()

  for j in range(NJ):
    pltpu.async_copy(onesv, acc.at[dstv.at[j]], sems, add=True)
  for j in range(NJ):
    pltpu.make_async_copy(onesv, acc.at[dstv.at[j]], sems).wait()
  plsc.subcore_barrier()

  pltpu.sync_copy(acc.at[pl.ds(s * ZROWS, ZROWS)], buf)
  pltpu.sync_copy(buf, out_h.at[c, pl.ds(s * ZROWS, ZROWS)])


def _make_deg():
  return pl.kernel(
      _deg_body,
      out_type=jax.ShapeDtypeStruct((NUM_CORES, NACC, C), jnp.float32),
      mesh=_sc_mesh(),
      scratch_types=[
          pltpu.VMEM((NJ, ROWS), jnp.int32),
          pltpu.VMEM((ROWS, C), jnp.float32),
          pltpu.VMEM((ZROWS, C), jnp.float32),
          pltpu.VMEM_SHARED((NACC, C), jnp.float32),
          pltpu.SemaphoreType.DMA,
      ],
      compiler_params=_SC_PARAMS,
      name="gcn_degree",
  )


def _tc2_body(degp_ref, x_ref, w1_ref, dinv_ref, g1_ref):
  deg = degp_ref[0] + degp_ref[1] + 1.0
  dinv = lax.rsqrt(deg)
  dinv_ref[...] = dinv
  h = jnp.dot(x_ref[...], w1_ref[...], preferred_element_type=jnp.float32)
  g1_ref[...] = h * dinv


def _tc3_body(s1p_ref, g1_ref, dinv_ref, b1_ref, w2_ref, g2_ref):
  dinv = dinv_ref[...]
  ssum = s1p_ref[0] + s1p_ref[1] + g1_ref[...]
  z = jnp.maximum(dinv * ssum + b1_ref[...], 0.0)
  h2 = jnp.dot(z, w2_ref[...], preferred_element_type=jnp.float32)
  g2_ref[...] = h2 * dinv


def _tc4_body(s2p_ref, g2_ref, dinv_ref, b2_ref, wfc_ref, bfc_ref, y_ref):
  dinv = dinv_ref[...]
  ssum = s2p_ref[0] + s2p_ref[1] + g2_ref[...]
  z = jnp.maximum(dinv * ssum + b2_ref[...], 0.0)
  y = jnp.dot(z, wfc_ref[...], preferred_element_type=jnp.float32)
  y_ref[...] = jax.nn.sigmoid(y + bfc_ref[...])


def kernel(x, edge_index, W1, b1, W2, b2, Wfc, bfc):
  f32 = jnp.float32
  ei = edge_index.astype(jnp.int32)
  npad = NW * EPT - N_EDGES
  src = jnp.concatenate([ei[0], jnp.zeros((npad,), jnp.int32)])
  dst = jnp.concatenate([ei[1], jnp.full((npad,), TRASH, jnp.int32)])
  src = src.reshape(NW, NJ, ROWS)
  dst = dst.reshape(NW, NJ, ROWS)

  W1p = jnp.pad(W1, ((0, 0), (0, C - 6))).astype(f32)
  b1p = jnp.pad(b1, (0, C - 6)).reshape(1, C)
  W2p = jnp.pad(W2, ((0, C - 6), (0, C - 4))).astype(f32)
  b2p = jnp.pad(b2, (0, C - 4)).reshape(1, C)
  Wfcp = jnp.pad(Wfc, ((0, C - 4), (0, 0))).astype(f32)

  ones_rows = jnp.ones((ROWS, C), f32)
  zrows = jnp.zeros((ZROWS, C), f32)

  degp = _make_deg()(dst, ones_rows, zrows)
  degp = degp[:, :N_NODES, 0:1]

  dinv, g1 = pl.pallas_call(
      _tc2_body,
      out_shape=[jax.ShapeDtypeStruct((N_NODES, 1), f32),
                 jax.ShapeDtypeStruct((N_NODES, C), f32)],
  )(degp, x, W1p)

  s1p = _make_scatter()(src, dst, g1, zrows)
  s1p = s1p[:, :N_NODES, :]

  g2 = pl.pallas_call(
      _tc3_body,
      out_shape=jax.ShapeDtypeStruct((N_NODES, C), f32),
  )(s1p, g1, dinv, b1p, W2p)

  s2p = _make_scatter()(src, dst, g2, zrows)
  s2p = s2p[:, :N_NODES, :]

  y = pl.pallas_call(
      _tc4_body,
      out_shape=jax.ShapeDtypeStruct((N_NODES, 1), f32),
  )(s2p, g2, dinv, b2p, Wfcp, bfc.reshape(1, 1))
  return y


# speedup vs baseline: 39.0436x; 1.1029x over previous
import functools

import jax
import jax.numpy as jnp
from jax import lax
from jax.experimental import pallas as pl
from jax.experimental.pallas import tpu as pltpu
from jax.experimental.pallas import tpu_sc as plsc

N_NODES = 10000
N_EDGES = 320000
NUM_CORES = 2
NUM_SUBCORES = 16
NW = NUM_CORES * NUM_SUBCORES
EPT = 10240
ROWS = 128
NJ = EPT // ROWS
NACC = 10240
ZROWS = NACC // NUM_SUBCORES
TRASH = 10100
C = 8


def _sc_mesh():
  return plsc.VectorSubcoreMesh(core_axis_name="c", subcore_axis_name="s",
                                num_cores=NUM_CORES, num_subcores=NUM_SUBCORES)


_SC_PARAMS = pltpu.CompilerParams(use_tc_tiling_on_sc=False)


def _scatter_body(src_h, dst_h, table_h, zeros_h, out_h,
                  srcv, dstv, msg, buf, acc, semg, sems):
  c = lax.axis_index("c")
  s = lax.axis_index("s")
  wid = c * NUM_SUBCORES + s

  pltpu.sync_copy(src_h.at[wid], srcv)
  pltpu.sync_copy(dst_h.at[wid], dstv)

  pltpu.sync_copy(zeros_h, buf)
  pltpu.sync_copy(buf, acc.at[pl.ds(s * ZROWS, ZROWS)])
  plsc.subcore_barrier()

  pltpu.async_copy(table_h.at[srcv], msg, semg).wait()

  for j in range(NJ):
    pltpu.async_copy(msg.at[pl.ds(j * ROWS, ROWS)], acc.at[dstv.at[j]],
                     sems, add=True)
  for j in range(NJ):
    pltpu.make_async_copy(msg.at[pl.ds(j * ROWS, ROWS)],
                          acc.at[dstv.at[j]], sems).wait()
  plsc.subcore_barrier()

  pltpu.sync_copy(acc.at[pl.ds(s * ZROWS, ZROWS)], buf)
  pltpu.sync_copy(buf, out_h.at[c, pl.ds(s * ZROWS, ZROWS)])


def _make_scatter():
  return pl.kernel(
      _scatter_body,
      out_type=jax.ShapeDtypeStruct((NUM_CORES, NACC, C), jnp.float32),
      mesh=_sc_mesh(),
      scratch_types=[
          pltpu.VMEM((EPT,), jnp.int32),
          pltpu.VMEM((NJ, ROWS), jnp.int32),
          pltpu.VMEM((EPT, C), jnp.float32),
          pltpu.VMEM((ZROWS, C), jnp.float32),
          pltpu.VMEM_SHARED((NACC, C), jnp.float32),
          pltpu.SemaphoreType.DMA,
          pltpu.SemaphoreType.DMA,
      ],
      compiler_params=_SC_PARAMS,
      name="gcn_scatter",
  )


def _deg_body(dst_h, ones_h, zeros_h, out_h, dstv, onesv, buf, acc, sems):
  c = lax.axis_index("c")
  s = lax.axis_index("s")
  wid = c * NUM_SUBCORES + s

  pltpu.sync_copy(dst_h.at[wid], dstv)
  pltpu.sync_copy(ones_h, onesv)
  pltpu.sync_copy(zeros_h, buf)
  pltpu.sync_copy(buf, acc.at[pl.ds(s * ZROWS, ZROWS)])
  plsc.subcore_barrier()

  for j in range(NJ):
    pltpu.async_copy(onesv, acc.at[dstv.at[j]], sems, add=True)
  for j in range(NJ):
    pltpu.make_async_copy(onesv, acc.at[dstv.at[j]], sems).wait()
  plsc.subcore_barrier()

  pltpu.sync_copy(acc.at[pl.ds(s * ZROWS, ZROWS)], buf)
  pltpu.sync_copy(buf, out_h.at[c, pl.ds(s * ZROWS, ZROWS)])


def _make_deg():
  return pl.kernel(
      _deg_body,
      out_type=jax.ShapeDtypeStruct((NUM_CORES, NACC, C), jnp.float32),
      mesh=_sc_mesh(),
      scratch_types=[
          pltpu.VMEM((NJ, ROWS), jnp.int32),
          pltpu.VMEM((ROWS, C), jnp.float32),
          pltpu.VMEM((ZROWS, C), jnp.float32),
          pltpu.VMEM_SHARED((NACC, C), jnp.float32),
          pltpu.SemaphoreType.DMA,
      ],
      compiler_params=_SC_PARAMS,
      name="gcn_degree",
  )


def _tc2_body(degp_ref, x_ref, w1_ref, dinv_ref, g1_ref):
  deg = (degp_ref[0, :N_NODES, 0:1] + degp_ref[1, :N_NODES, 0:1]
         + 1.0)
  dinv = lax.rsqrt(deg)
  dinv_ref[...] = dinv
  h = jnp.dot(x_ref[...], w1_ref[...], preferred_element_type=jnp.float32)
  g1_ref[...] = h * dinv


def _tc3_body(s1p_ref, g1_ref, dinv_ref, b1_ref, w2_ref, g2_ref):
  dinv = dinv_ref[...]
  ssum = s1p_ref[0, :N_NODES] + s1p_ref[1, :N_NODES] + g1_ref[...]
  z = jnp.maximum(dinv * ssum + b1_ref[...], 0.0)
  h2 = jnp.dot(z, w2_ref[...], preferred_element_type=jnp.float32)
  g2_ref[...] = h2 * dinv


def _tc4_body(s2p_ref, g2_ref, dinv_ref, b2_ref, wfc_ref, bfc_ref, y_ref):
  dinv = dinv_ref[...]
  ssum = s2p_ref[0, :N_NODES] + s2p_ref[1, :N_NODES] + g2_ref[...]
  z = jnp.maximum(dinv * ssum + b2_ref[...], 0.0)
  y = jnp.dot(z, wfc_ref[...], preferred_element_type=jnp.float32)
  y_ref[...] = jax.nn.sigmoid(y + bfc_ref[...])


def kernel(x, edge_index, W1, b1, W2, b2, Wfc, bfc):
  f32 = jnp.float32
  ei = edge_index.astype(jnp.int32)
  npad = NW * EPT - N_EDGES
  src = jnp.concatenate([ei[0], jnp.zeros((npad,), jnp.int32)])
  dst = jnp.concatenate([ei[1], jnp.full((npad,), TRASH, jnp.int32)])
  src = src.reshape(NW, EPT)
  dst = dst.reshape(NW, NJ, ROWS)

  W1p = jnp.pad(W1, ((0, 0), (0, C - 6))).astype(f32)
  b1p = jnp.pad(b1, (0, C - 6)).reshape(1, C)
  W2p = jnp.pad(W2, ((0, C - 6), (0, C - 4))).astype(f32)
  b2p = jnp.pad(b2, (0, C - 4)).reshape(1, C)
  Wfcp = jnp.pad(Wfc, ((0, C - 4), (0, 0))).astype(f32)

  ones_rows = jnp.ones((ROWS, C), f32)
  zrows = jnp.zeros((ZROWS, C), f32)

  degp = _make_deg()(dst, ones_rows, zrows)

  dinv, g1 = pl.pallas_call(
      _tc2_body,
      out_shape=[jax.ShapeDtypeStruct((N_NODES, 1), f32),
                 jax.ShapeDtypeStruct((N_NODES, C), f32)],
  )(degp, x, W1p)

  s1p = _make_scatter()(src, dst, g1, zrows)

  g2 = pl.pallas_call(
      _tc3_body,
      out_shape=jax.ShapeDtypeStruct((N_NODES, C), f32),
  )(s1p, g1, dinv, b1p, W2p)

  s2p = _make_scatter()(src, dst, g2, zrows)

  y = pl.pallas_call(
      _tc4_body,
      out_shape=jax.ShapeDtypeStruct((N_NODES, 1), f32),
  )(s2p, g2, dinv, b2p, Wfcp, bfc.reshape(1, 1))
  return y


# speedup vs baseline: 54.1356x; 1.3865x over previous
import functools

import jax
import jax.numpy as jnp
from jax import lax
from jax.experimental import pallas as pl
from jax.experimental.pallas import tpu as pltpu
from jax.experimental.pallas import tpu_sc as plsc

N_NODES = 10000
N_EDGES = 320000
NUM_CORES = 2
NUM_SUBCORES = 16
NW = NUM_CORES * NUM_SUBCORES
EPT = 10240
ROWS = 128
NJ = EPT // ROWS
NACC = 10240
ZROWS = NACC // NUM_SUBCORES
TRASH = 10100
C = 8
TROWS = N_NODES // NUM_SUBCORES


def _sc_mesh():
  return plsc.VectorSubcoreMesh(core_axis_name="c", subcore_axis_name="s",
                                num_cores=NUM_CORES, num_subcores=NUM_SUBCORES)


_SC_PARAMS = pltpu.CompilerParams(use_tc_tiling_on_sc=False)


def _scatter_body(src_h, dst_h, table_h, zeros_h, out_h,
                  srcv, dstv, msg, buf, tab_s, acc, semg, sems):
  c = lax.axis_index("c")
  s = lax.axis_index("s")
  wid = c * NUM_SUBCORES + s

  pltpu.sync_copy(src_h.at[wid], srcv)
  pltpu.sync_copy(dst_h.at[wid], dstv)

  pltpu.sync_copy(table_h.at[pl.ds(s * TROWS, TROWS)], buf.at[pl.ds(0, TROWS)])
  pltpu.sync_copy(buf.at[pl.ds(0, TROWS)], tab_s.at[pl.ds(s * TROWS, TROWS)])
  pltpu.sync_copy(zeros_h, buf)
  pltpu.sync_copy(buf, acc.at[pl.ds(s * ZROWS, ZROWS)])
  plsc.subcore_barrier()

  pltpu.async_copy(tab_s.at[srcv], msg, semg).wait()

  for j in range(NJ):
    pltpu.async_copy(msg.at[pl.ds(j * ROWS, ROWS)], acc.at[dstv.at[j]],
                     sems, add=True)
  for j in range(NJ):
    pltpu.make_async_copy(msg.at[pl.ds(j * ROWS, ROWS)],
                          acc.at[dstv.at[j]], sems).wait()
  plsc.subcore_barrier()

  pltpu.sync_copy(acc.at[pl.ds(s * ZROWS, ZROWS)], buf)
  pltpu.sync_copy(buf, out_h.at[c, pl.ds(s * ZROWS, ZROWS)])


def _make_scatter():
  return pl.kernel(
      _scatter_body,
      out_type=jax.ShapeDtypeStruct((NUM_CORES, NACC, C), jnp.float32),
      mesh=_sc_mesh(),
      scratch_types=[
          pltpu.VMEM((EPT,), jnp.int32),
          pltpu.VMEM((NJ, ROWS), jnp.int32),
          pltpu.VMEM((EPT, C), jnp.float32),
          pltpu.VMEM((ZROWS, C), jnp.float32),
          pltpu.VMEM_SHARED((N_NODES, C), jnp.float32),
          pltpu.VMEM_SHARED((NACC, C), jnp.float32),
          pltpu.SemaphoreType.DMA,
          pltpu.SemaphoreType.DMA,
      ],
      compiler_params=_SC_PARAMS,
      name="gcn_scatter",
  )


def _deg_body(dst_h, ones_h, zeros_h, out_h, dstv, onesv, buf, acc, sems):
  c = lax.axis_index("c")
  s = lax.axis_index("s")
  wid = c * NUM_SUBCORES + s

  pltpu.sync_copy(dst_h.at[wid], dstv)
  pltpu.sync_copy(ones_h, onesv)
  pltpu.sync_copy(zeros_h, buf)
  pltpu.sync_copy(buf, acc.at[pl.ds(s * ZROWS, ZROWS)])
  plsc.subcore_barrier()

  for j in range(NJ):
    pltpu.async_copy(onesv, acc.at[dstv.at[j]], sems, add=True)
  for j in range(NJ):
    pltpu.make_async_copy(onesv, acc.at[dstv.at[j]], sems).wait()
  plsc.subcore_barrier()

  pltpu.sync_copy(acc.at[pl.ds(s * ZROWS, ZROWS)], buf)
  pltpu.sync_copy(buf, out_h.at[c, pl.ds(s * ZROWS, ZROWS)])


def _make_deg():
  return pl.kernel(
      _deg_body,
      out_type=jax.ShapeDtypeStruct((NUM_CORES, NACC, C), jnp.float32),
      mesh=_sc_mesh(),
      scratch_types=[
          pltpu.VMEM((NJ, ROWS), jnp.int32),
          pltpu.VMEM((ROWS, C), jnp.float32),
          pltpu.VMEM((ZROWS, C), jnp.float32),
          pltpu.VMEM_SHARED((NACC, C), jnp.float32),
          pltpu.SemaphoreType.DMA,
      ],
      compiler_params=_SC_PARAMS,
      name="gcn_degree",
  )


def _tc2_body(degp_ref, x_ref, w1_ref, dinv_ref, g1_ref):
  deg = (degp_ref[0, :N_NODES, 0:1] + degp_ref[1, :N_NODES, 0:1]
         + 1.0)
  dinv = lax.rsqrt(deg)
  dinv_ref[...] = dinv
  h = jnp.dot(x_ref[...], w1_ref[...], preferred_element_type=jnp.float32)
  g1_ref[...] = h * dinv


def _tc3_body(s1p_ref, g1_ref, dinv_ref, b1_ref, w2_ref, g2_ref):
  dinv = dinv_ref[...]
  ssum = s1p_ref[0, :N_NODES] + s1p_ref[1, :N_NODES] + g1_ref[...]
  z = jnp.maximum(dinv * ssum + b1_ref[...], 0.0)
  h2 = jnp.dot(z, w2_ref[...], preferred_element_type=jnp.float32)
  g2_ref[...] = h2 * dinv


def _tc4_body(s2p_ref, g2_ref, dinv_ref, b2_ref, wfc_ref, bfc_ref, y_ref):
  dinv = dinv_ref[...]
  ssum = s2p_ref[0, :N_NODES] + s2p_ref[1, :N_NODES] + g2_ref[...]
  z = jnp.maximum(dinv * ssum + b2_ref[...], 0.0)
  y = jnp.dot(z, wfc_ref[...], preferred_element_type=jnp.float32)
  y_ref[...] = jax.nn.sigmoid(y + bfc_ref[...])


def kernel(x, edge_index, W1, b1, W2, b2, Wfc, bfc):
  f32 = jnp.float32
  ei = edge_index.astype(jnp.int32)
  npad = NW * EPT - N_EDGES
  src = jnp.concatenate([ei[0], jnp.zeros((npad,), jnp.int32)])
  dst = jnp.concatenate([ei[1], jnp.full((npad,), TRASH, jnp.int32)])
  src = src.reshape(NW, EPT)
  dst = dst.reshape(NW, NJ, ROWS)

  W1p = jnp.pad(W1, ((0, 0), (0, C - 6))).astype(f32)
  b1p = jnp.pad(b1, (0, C - 6)).reshape(1, C)
  W2p = jnp.pad(W2, ((0, C - 6), (0, C - 4))).astype(f32)
  b2p = jnp.pad(b2, (0, C - 4)).reshape(1, C)
  Wfcp = jnp.pad(Wfc, ((0, C - 4), (0, 0))).astype(f32)

  ones_rows = jnp.ones((ROWS, C), f32)
  zrows = jnp.zeros((ZROWS, C), f32)

  degp = _make_deg()(dst, ones_rows, zrows)

  dinv, g1 = pl.pallas_call(
      _tc2_body,
      out_shape=[jax.ShapeDtypeStruct((N_NODES, 1), f32),
                 jax.ShapeDtypeStruct((N_NODES, C), f32)],
  )(degp, x, W1p)

  s1p = _make_scatter()(src, dst, g1, zrows)

  g2 = pl.pallas_call(
      _tc3_body,
      out_shape=jax.ShapeDtypeStruct((N_NODES, C), f32),
  )(s1p, g1, dinv, b1p, W2p)

  s2p = _make_scatter()(src, dst, g2, zrows)

  y = pl.pallas_call(
      _tc4_body,
      out_shape=jax.ShapeDtypeStruct((N_NODES, 1), f32),
  )(s2p, g2, dinv, b2p, Wfcp, bfc.reshape(1, 1))
  return y


# speedup vs baseline: 58.6266x; 1.0830x over previous
import functools

import jax
import jax.numpy as jnp
from jax import lax
from jax.experimental import pallas as pl
from jax.experimental.pallas import tpu as pltpu
from jax.experimental.pallas import tpu_sc as plsc

N_NODES = 10000
N_EDGES = 320000
NUM_CORES = 2
NUM_SUBCORES = 16
NW = NUM_CORES * NUM_SUBCORES
EPT = 10240
ROWS = 128
NJ = EPT // ROWS
NACC = 10240
ZROWS = NACC // NUM_SUBCORES
TRASH = 10100
C = 8
TROWS = N_NODES // NUM_SUBCORES
NB = 8
GROWS = EPT // NB


def _sc_mesh():
  return plsc.VectorSubcoreMesh(core_axis_name="c", subcore_axis_name="s",
                                num_cores=NUM_CORES, num_subcores=NUM_SUBCORES)


_SC_PARAMS = pltpu.CompilerParams(use_tc_tiling_on_sc=False)


def _scatter_body(src_h, dst_h, table_h, zeros_h, out_h,
                  srcv, dstv, msg, buf, tab_s, acc, semg, sems, semi):
  c = lax.axis_index("c")
  s = lax.axis_index("s")
  wid = c * NUM_SUBCORES + s

  cp_src = pltpu.async_copy(src_h.at[wid], srcv, semi)
  cp_dst = pltpu.async_copy(dst_h.at[wid], dstv, semi)

  pltpu.sync_copy(table_h.at[pl.ds(s * TROWS, TROWS)], buf.at[pl.ds(0, TROWS)])
  pltpu.sync_copy(buf.at[pl.ds(0, TROWS)], tab_s.at[pl.ds(s * TROWS, TROWS)])
  pltpu.sync_copy(zeros_h, buf)
  pltpu.sync_copy(buf, acc.at[pl.ds(s * ZROWS, ZROWS)])
  cp_src.wait()
  cp_dst.wait()
  plsc.subcore_barrier()

  for b in range(NB):
    pltpu.async_copy(tab_s.at[srcv.at[pl.ds(b * GROWS, GROWS)]],
                     msg.at[pl.ds(b * GROWS, GROWS)], semg)
  for b in range(NB):
    pltpu.make_async_copy(tab_s.at[srcv.at[pl.ds(b * GROWS, GROWS)]],
                          msg.at[pl.ds(b * GROWS, GROWS)], semg).wait()
    for jj in range(GROWS // ROWS):
      j = b * (GROWS // ROWS) + jj
      pltpu.async_copy(msg.at[pl.ds(j * ROWS, ROWS)], acc.at[dstv.at[j]],
                       sems, add=True)
  for j in range(NJ):
    pltpu.make_async_copy(msg.at[pl.ds(j * ROWS, ROWS)],
                          acc.at[dstv.at[j]], sems).wait()
  plsc.subcore_barrier()

  pltpu.sync_copy(acc.at[pl.ds(s * ZROWS, ZROWS)], buf)
  pltpu.sync_copy(buf, out_h.at[c, pl.ds(s * ZROWS, ZROWS)])


def _make_scatter():
  return pl.kernel(
      _scatter_body,
      out_type=jax.ShapeDtypeStruct((NUM_CORES, NACC, C), jnp.float32),
      mesh=_sc_mesh(),
      scratch_types=[
          pltpu.VMEM((EPT,), jnp.int32),
          pltpu.VMEM((NJ, ROWS), jnp.int32),
          pltpu.VMEM((EPT, C), jnp.float32),
          pltpu.VMEM((ZROWS, C), jnp.float32),
          pltpu.VMEM_SHARED((N_NODES, C), jnp.float32),
          pltpu.VMEM_SHARED((NACC, C), jnp.float32),
          pltpu.SemaphoreType.DMA,
          pltpu.SemaphoreType.DMA,
          pltpu.SemaphoreType.DMA,
      ],
      compiler_params=_SC_PARAMS,
      name="gcn_scatter",
  )


def _deg_body(dst_h, ones_h, zeros_h, out_h, dstv, onesv, buf, acc, sems):
  c = lax.axis_index("c")
  s = lax.axis_index("s")
  wid = c * NUM_SUBCORES + s

  pltpu.sync_copy(dst_h.at[wid], dstv)
  pltpu.sync_copy(ones_h, onesv)
  pltpu.sync_copy(zeros_h, buf)
  pltpu.sync_copy(buf, acc.at[pl.ds(s * ZROWS, ZROWS)])
  plsc.subcore_barrier()

  for j in range(NJ):
    pltpu.async_copy(onesv, acc.at[dstv.at[j]], sems, add=True)
  for j in range(NJ):
    pltpu.make_async_copy(onesv, acc.at[dstv.at[j]], sems).wait()
  plsc.subcore_barrier()

  pltpu.sync_copy(acc.at[pl.ds(s * ZROWS, ZROWS)], buf)
  pltpu.sync_copy(buf, out_h.at[c, pl.ds(s * ZROWS, ZROWS)])


def _make_deg():
  return pl.kernel(
      _deg_body,
      out_type=jax.ShapeDtypeStruct((NUM_CORES, NACC, C), jnp.float32),
      mesh=_sc_mesh(),
      scratch_types=[
          pltpu.VMEM((NJ, ROWS), jnp.int32),
          pltpu.VMEM((ROWS, C), jnp.float32),
          pltpu.VMEM((ZROWS, C), jnp.float32),
          pltpu.VMEM_SHARED((NACC, C), jnp.float32),
          pltpu.SemaphoreType.DMA,
      ],
      compiler_params=_SC_PARAMS,
      name="gcn_degree",
  )


def _tc2_body(degp_ref, x_ref, w1_ref, dinv_ref, g1_ref):
  deg = (degp_ref[0, :N_NODES, 0:1] + degp_ref[1, :N_NODES, 0:1]
         + 1.0)
  dinv = lax.rsqrt(deg)
  dinv_ref[...] = dinv
  h = jnp.dot(x_ref[...], w1_ref[...], preferred_element_type=jnp.float32)
  g1_ref[...] = h * dinv


def _tc3_body(s1p_ref, g1_ref, dinv_ref, b1_ref, w2_ref, g2_ref):
  dinv = dinv_ref[...]
  ssum = s1p_ref[0, :N_NODES] + s1p_ref[1, :N_NODES] + g1_ref[...]
  z = jnp.maximum(dinv * ssum + b1_ref[...], 0.0)
  h2 = jnp.dot(z, w2_ref[...], preferred_element_type=jnp.float32)
  g2_ref[...] = h2 * dinv


def _tc4_body(s2p_ref, g2_ref, dinv_ref, b2_ref, wfc_ref, bfc_ref, y_ref):
  dinv = dinv_ref[...]
  ssum = s2p_ref[0, :N_NODES] + s2p_ref[1, :N_NODES] + g2_ref[...]
  z = jnp.maximum(dinv * ssum + b2_ref[...], 0.0)
  y = jnp.dot(z, wfc_ref[...], preferred_element_type=jnp.float32)
  y_ref[...] = jax.nn.sigmoid(y + bfc_ref[...])


def kernel(x, edge_index, W1, b1, W2, b2, Wfc, bfc):
  f32 = jnp.float32
  ei = edge_index.astype(jnp.int32)
  npad = NW * EPT - N_EDGES
  src = jnp.concatenate([ei[0], jnp.zeros((npad,), jnp.int32)])
  dst = jnp.concatenate([ei[1], jnp.full((npad,), TRASH, jnp.int32)])
  src = src.reshape(NW, EPT)
  dst = dst.reshape(NW, NJ, ROWS)

  W1p = jnp.pad(W1, ((0, 0), (0, C - 6))).astype(f32)
  b1p = jnp.pad(b1, (0, C - 6)).reshape(1, C)
  W2p = jnp.pad(W2, ((0, C - 6), (0, C - 4))).astype(f32)
  b2p = jnp.pad(b2, (0, C - 4)).reshape(1, C)
  Wfcp = jnp.pad(Wfc, ((0, C - 4), (0, 0))).astype(f32)

  ones_rows = jnp.ones((ROWS, C), f32)
  zrows = jnp.zeros((ZROWS, C), f32)

  degp = _make_deg()(dst, ones_rows, zrows)

  dinv, g1 = pl.pallas_call(
      _tc2_body,
      out_shape=[jax.ShapeDtypeStruct((N_NODES, 1), f32),
                 jax.ShapeDtypeStruct((N_NODES, C), f32)],
  )(degp, x, W1p)

  s1p = _make_scatter()(src, dst, g1, zrows)

  g2 = pl.pallas_call(
      _tc3_body,
      out_shape=jax.ShapeDtypeStruct((N_NODES, C), f32),
  )(s1p, g1, dinv, b1p, W2p)

  s2p = _make_scatter()(src, dst, g2, zrows)

  y = pl.pallas_call(
      _tc4_body,
      out_shape=jax.ShapeDtypeStruct((N_NODES, 1), f32),
  )(s2p, g2, dinv, b2p, Wfcp, bfc.reshape(1, 1))
  return y


# speedup vs baseline: 59.2744x; 1.0111x over previous
import functools

import jax
import jax.numpy as jnp
from jax import lax
from jax.experimental import pallas as pl
from jax.experimental.pallas import tpu as pltpu
from jax.experimental.pallas import tpu_sc as plsc

N_NODES = 10000
N_EDGES = 320000
NUM_CORES = 2
NUM_SUBCORES = 16
NW = NUM_CORES * NUM_SUBCORES
EPT = 10240
ROWS = 128
NJ = EPT // ROWS
NACC = 10240
ZROWS = NACC // NUM_SUBCORES
TRASH = 10100
C = 8
TROWS = N_NODES // NUM_SUBCORES
NB = 8
GROWS = EPT // NB


def _sc_mesh():
  return plsc.VectorSubcoreMesh(core_axis_name="c", subcore_axis_name="s",
                                num_cores=NUM_CORES, num_subcores=NUM_SUBCORES)


_SC_PARAMS = pltpu.CompilerParams(use_tc_tiling_on_sc=False)


def _scatter_body(src_h, dst_h, table_h, zeros_h, out_h,
                  srcv, dstv, msg, buf, tab_s, acc, semg, sems, semi):
  c = lax.axis_index("c")
  s = lax.axis_index("s")
  wid = c * NUM_SUBCORES + s

  cp_src = pltpu.async_copy(src_h.at[wid], srcv, semi)
  cp_dst = pltpu.async_copy(dst_h.at[wid], dstv, semi)

  pltpu.sync_copy(table_h.at[pl.ds(s * TROWS, TROWS)],
                  tab_s.at[pl.ds(s * TROWS, TROWS)])
  pltpu.sync_copy(zeros_h, acc.at[pl.ds(s * ZROWS, ZROWS)])
  cp_src.wait()
  cp_dst.wait()
  plsc.subcore_barrier()

  for b in range(NB):
    pltpu.async_copy(tab_s.at[srcv.at[pl.ds(b * GROWS, GROWS)]],
                     msg.at[pl.ds(b * GROWS, GROWS)], semg)
  for b in range(NB):
    pltpu.make_async_copy(tab_s.at[srcv.at[pl.ds(b * GROWS, GROWS)]],
                          msg.at[pl.ds(b * GROWS, GROWS)], semg).wait()
    for jj in range(GROWS // ROWS):
      j = b * (GROWS // ROWS) + jj
      pltpu.async_copy(msg.at[pl.ds(j * ROWS, ROWS)], acc.at[dstv.at[j]],
                       sems, add=True)
  for j in range(NJ):
    pltpu.make_async_copy(msg.at[pl.ds(j * ROWS, ROWS)],
                          acc.at[dstv.at[j]], sems).wait()
  plsc.subcore_barrier()

  pltpu.sync_copy(acc.at[pl.ds(s * ZROWS, ZROWS)],
                  out_h.at[c, pl.ds(s * ZROWS, ZROWS)])


def _make_scatter():
  return pl.kernel(
      _scatter_body,
      out_type=jax.ShapeDtypeStruct((NUM_CORES, NACC, C), jnp.float32),
      mesh=_sc_mesh(),
      scratch_types=[
          pltpu.VMEM((EPT,), jnp.int32),
          pltpu.VMEM((NJ, ROWS), jnp.int32),
          pltpu.VMEM((EPT, C), jnp.float32),
          pltpu.VMEM((ZROWS, C), jnp.float32),
          pltpu.VMEM_SHARED((N_NODES, C), jnp.float32),
          pltpu.VMEM_SHARED((NACC, C), jnp.float32),
          pltpu.SemaphoreType.DMA,
          pltpu.SemaphoreType.DMA,
          pltpu.SemaphoreType.DMA,
      ],
      compiler_params=_SC_PARAMS,
      name="gcn_scatter",
  )


def _deg_body(dst_h, ones_h, zeros_h, out_h, dstv, onesv, buf, acc, sems):
  c = lax.axis_index("c")
  s = lax.axis_index("s")
  wid = c * NUM_SUBCORES + s

  pltpu.sync_copy(dst_h.at[wid], dstv)
  pltpu.sync_copy(ones_h, onesv)
  pltpu.sync_copy(zeros_h, acc.at[pl.ds(s * ZROWS, ZROWS)])
  plsc.subcore_barrier()

  for j in range(NJ):
    pltpu.async_copy(onesv, acc.at[dstv.at[j]], sems, add=True)
  for j in range(NJ):
    pltpu.make_async_copy(onesv, acc.at[dstv.at[j]], sems).wait()
  plsc.subcore_barrier()

  pltpu.sync_copy(acc.at[pl.ds(s * ZROWS, ZROWS)],
                  out_h.at[c, pl.ds(s * ZROWS, ZROWS)])


def _make_deg():
  return pl.kernel(
      _deg_body,
      out_type=jax.ShapeDtypeStruct((NUM_CORES, NACC, C), jnp.float32),
      mesh=_sc_mesh(),
      scratch_types=[
          pltpu.VMEM((NJ, ROWS), jnp.int32),
          pltpu.VMEM((ROWS, C), jnp.float32),
          pltpu.VMEM((ZROWS, C), jnp.float32),
          pltpu.VMEM_SHARED((NACC, C), jnp.float32),
          pltpu.SemaphoreType.DMA,
      ],
      compiler_params=_SC_PARAMS,
      name="gcn_degree",
  )


def _tc2_body(degp_ref, x_ref, w1_ref, dinv_ref, g1_ref):
  deg = (degp_ref[0, :N_NODES, 0:1] + degp_ref[1, :N_NODES, 0:1]
         + 1.0)
  dinv = lax.rsqrt(deg)
  dinv_ref[...] = dinv
  h = jnp.dot(x_ref[...], w1_ref[...], preferred_element_type=jnp.float32)
  g1_ref[...] = h * dinv


def _tc3_body(s1p_ref, g1_ref, dinv_ref, b1_ref, w2_ref, g2_ref):
  dinv = dinv_ref[...]
  ssum = s1p_ref[0, :N_NODES] + s1p_ref[1, :N_NODES] + g1_ref[...]
  z = jnp.maximum(dinv * ssum + b1_ref[...], 0.0)
  h2 = jnp.dot(z, w2_ref[...], preferred_element_type=jnp.float32)
  g2_ref[...] = h2 * dinv


def _tc4_body(s2p_ref, g2_ref, dinv_ref, b2_ref, wfc_ref, bfc_ref, y_ref):
  dinv = dinv_ref[...]
  ssum = s2p_ref[0, :N_NODES] + s2p_ref[1, :N_NODES] + g2_ref[...]
  z = jnp.maximum(dinv * ssum + b2_ref[...], 0.0)
  y = jnp.dot(z, wfc_ref[...], preferred_element_type=jnp.float32)
  y_ref[...] = jax.nn.sigmoid(y + bfc_ref[...])


def kernel(x, edge_index, W1, b1, W2, b2, Wfc, bfc):
  f32 = jnp.float32
  ei = edge_index.astype(jnp.int32)
  npad = NW * EPT - N_EDGES
  src = jnp.concatenate([ei[0], jnp.zeros((npad,), jnp.int32)])
  dst = jnp.concatenate([ei[1], jnp.full((npad,), TRASH, jnp.int32)])
  src = src.reshape(NW, EPT)
  dst = dst.reshape(NW, NJ, ROWS)

  W1p = jnp.pad(W1, ((0, 0), (0, C - 6))).astype(f32)
  b1p = jnp.pad(b1, (0, C - 6)).reshape(1, C)
  W2p = jnp.pad(W2, ((0, C - 6), (0, C - 4))).astype(f32)
  b2p = jnp.pad(b2, (0, C - 4)).reshape(1, C)
  Wfcp = jnp.pad(Wfc, ((0, C - 4), (0, 0))).astype(f32)

  ones_rows = jnp.ones((ROWS, C), f32)
  zrows = jnp.zeros((ZROWS, C), f32)

  degp = _make_deg()(dst, ones_rows, zrows)

  dinv, g1 = pl.pallas_call(
      _tc2_body,
      out_shape=[jax.ShapeDtypeStruct((N_NODES, 1), f32),
                 jax.ShapeDtypeStruct((N_NODES, C), f32)],
  )(degp, x, W1p)

  s1p = _make_scatter()(src, dst, g1, zrows)

  g2 = pl.pallas_call(
      _tc3_body,
      out_shape=jax.ShapeDtypeStruct((N_NODES, C), f32),
  )(s1p, g1, dinv, b1p, W2p)

  s2p = _make_scatter()(src, dst, g2, zrows)

  y = pl.pallas_call(
      _tc4_body,
      out_shape=jax.ShapeDtypeStruct((N_NODES, 1), f32),
  )(s2p, g2, dinv, b2p, Wfcp, bfc.reshape(1, 1))
  return y


# speedup vs baseline: 59.7192x; 1.0075x over previous
import functools

import jax
import jax.numpy as jnp
from jax import lax
from jax.experimental import pallas as pl
from jax.experimental.pallas import tpu as pltpu
from jax.experimental.pallas import tpu_sc as plsc

N_NODES = 10000
N_EDGES = 320000
NUM_CORES = 2
NUM_SUBCORES = 16
NW = NUM_CORES * NUM_SUBCORES
EPT = 10240
ROWS = 128
NJ = EPT // ROWS
NACC = 10240
ZROWS = NACC // NUM_SUBCORES
TRASH = 10100
C = 8
TROWS = N_NODES // NUM_SUBCORES
NB = 16
GROWS = EPT // NB


def _sc_mesh():
  return plsc.VectorSubcoreMesh(core_axis_name="c", subcore_axis_name="s",
                                num_cores=NUM_CORES, num_subcores=NUM_SUBCORES)


_SC_PARAMS = pltpu.CompilerParams(use_tc_tiling_on_sc=False)


def _scatter_body(src_h, dst_h, table_h, zeros_h, out_h,
                  srcv, dstv, msg, buf, tab_s, acc, semg, sems, semi):
  c = lax.axis_index("c")
  s = lax.axis_index("s")
  wid = c * NUM_SUBCORES + s

  cp_src = pltpu.async_copy(src_h.at[wid], srcv, semi)
  cp_dst = pltpu.async_copy(dst_h.at[wid], dstv, semi)

  pltpu.sync_copy(table_h.at[pl.ds(s * TROWS, TROWS)],
                  tab_s.at[pl.ds(s * TROWS, TROWS)])
  pltpu.sync_copy(zeros_h, acc.at[pl.ds(s * ZROWS, ZROWS)])
  cp_src.wait()
  cp_dst.wait()
  plsc.subcore_barrier()

  for b in range(NB):
    pltpu.async_copy(tab_s.at[srcv.at[pl.ds(b * GROWS, GROWS)]],
                     msg.at[pl.ds(b * GROWS, GROWS)], semg)
  for b in range(NB):
    pltpu.make_async_copy(tab_s.at[srcv.at[pl.ds(b * GROWS, GROWS)]],
                          msg.at[pl.ds(b * GROWS, GROWS)], semg).wait()
    for jj in range(GROWS // ROWS):
      j = b * (GROWS // ROWS) + jj
      pltpu.async_copy(msg.at[pl.ds(j * ROWS, ROWS)], acc.at[dstv.at[j]],
                       sems, add=True)
  for j in range(NJ):
    pltpu.make_async_copy(msg.at[pl.ds(j * ROWS, ROWS)],
                          acc.at[dstv.at[j]], sems).wait()
  plsc.subcore_barrier()

  pltpu.sync_copy(acc.at[pl.ds(s * ZROWS, ZROWS)],
                  out_h.at[c, pl.ds(s * ZROWS, ZROWS)])


def _make_scatter():
  return pl.kernel(
      _scatter_body,
      out_type=jax.ShapeDtypeStruct((NUM_CORES, NACC, C), jnp.float32),
      mesh=_sc_mesh(),
      scratch_types=[
          pltpu.VMEM((EPT,), jnp.int32),
          pltpu.VMEM((NJ, ROWS), jnp.int32),
          pltpu.VMEM((EPT, C), jnp.float32),
          pltpu.VMEM((ZROWS, C), jnp.float32),
          pltpu.VMEM_SHARED((N_NODES, C), jnp.float32),
          pltpu.VMEM_SHARED((NACC, C), jnp.float32),
          pltpu.SemaphoreType.DMA,
          pltpu.SemaphoreType.DMA,
          pltpu.SemaphoreType.DMA,
      ],
      compiler_params=_SC_PARAMS,
      name="gcn_scatter",
  )


def _deg_body(dst_h, ones_h, zeros_h, out_h, dstv, onesv, buf, acc, sems):
  c = lax.axis_index("c")
  s = lax.axis_index("s")
  wid = c * NUM_SUBCORES + s

  pltpu.sync_copy(dst_h.at[wid], dstv)
  pltpu.sync_copy(ones_h, onesv)
  pltpu.sync_copy(zeros_h, acc.at[pl.ds(s * ZROWS, ZROWS)])
  plsc.subcore_barrier()

  for j in range(NJ):
    pltpu.async_copy(onesv, acc.at[dstv.at[j]], sems, add=True)
  for j in range(NJ):
    pltpu.make_async_copy(onesv, acc.at[dstv.at[j]], sems).wait()
  plsc.subcore_barrier()

  pltpu.sync_copy(acc.at[pl.ds(s * ZROWS, ZROWS)],
                  out_h.at[c, pl.ds(s * ZROWS, ZROWS)])


def _make_deg():
  return pl.kernel(
      _deg_body,
      out_type=jax.ShapeDtypeStruct((NUM_CORES, NACC, C), jnp.float32),
      mesh=_sc_mesh(),
      scratch_types=[
          pltpu.VMEM((NJ, ROWS), jnp.int32),
          pltpu.VMEM((ROWS, C), jnp.float32),
          pltpu.VMEM((ZROWS, C), jnp.float32),
          pltpu.VMEM_SHARED((NACC, C), jnp.float32),
          pltpu.SemaphoreType.DMA,
      ],
      compiler_params=_SC_PARAMS,
      name="gcn_degree",
  )


def _tc2_body(degp_ref, x_ref, w1_ref, dinv_ref, g1_ref):
  deg = (degp_ref[0, :N_NODES, 0:1] + degp_ref[1, :N_NODES, 0:1]
         + 1.0)
  dinv = lax.rsqrt(deg)
  dinv_ref[...] = dinv
  h = jnp.dot(x_ref[...], w1_ref[...], preferred_element_type=jnp.float32)
  g1_ref[...] = h * dinv


def _tc3_body(s1p_ref, g1_ref, dinv_ref, b1_ref, w2_ref, g2_ref):
  dinv = dinv_ref[...]
  ssum = s1p_ref[0, :N_NODES] + s1p_ref[1, :N_NODES] + g1_ref[...]
  z = jnp.maximum(dinv * ssum + b1_ref[...], 0.0)
  h2 = jnp.dot(z, w2_ref[...], preferred_element_type=jnp.float32)
  g2_ref[...] = h2 * dinv


def _tc4_body(s2p_ref, g2_ref, dinv_ref, b2_ref, wfc_ref, bfc_ref, y_ref):
  dinv = dinv_ref[...]
  ssum = s2p_ref[0, :N_NODES] + s2p_ref[1, :N_NODES] + g2_ref[...]
  z = jnp.maximum(dinv * ssum + b2_ref[...], 0.0)
  y = jnp.dot(z, wfc_ref[...], preferred_element_type=jnp.float32)
  y_ref[...] = jax.nn.sigmoid(y + bfc_ref[...])


def kernel(x, edge_index, W1, b1, W2, b2, Wfc, bfc):
  f32 = jnp.float32
  ei = edge_index.astype(jnp.int32)
  npad = NW * EPT - N_EDGES
  src = jnp.concatenate([ei[0], jnp.zeros((npad,), jnp.int32)])
  dst = jnp.concatenate([ei[1], jnp.full((npad,), TRASH, jnp.int32)])
  src = src.reshape(NW, EPT)
  dst = dst.reshape(NW, NJ, ROWS)

  W1p = jnp.pad(W1, ((0, 0), (0, C - 6))).astype(f32)
  b1p = jnp.pad(b1, (0, C - 6)).reshape(1, C)
  W2p = jnp.pad(W2, ((0, C - 6), (0, C - 4))).astype(f32)
  b2p = jnp.pad(b2, (0, C - 4)).reshape(1, C)
  Wfcp = jnp.pad(Wfc, ((0, C - 4), (0, 0))).astype(f32)

  ones_rows = jnp.ones((ROWS, C), f32)
  zrows = jnp.zeros((ZROWS, C), f32)

  degp = _make_deg()(dst, ones_rows, zrows)

  dinv, g1 = pl.pallas_call(
      _tc2_body,
      out_shape=[jax.ShapeDtypeStruct((N_NODES, 1), f32),
                 jax.ShapeDtypeStruct((N_NODES, C), f32)],
  )(degp, x, W1p)

  s1p = _make_scatter()(src, dst, g1, zrows)

  g2 = pl.pallas_call(
      _tc3_body,
      out_shape=jax.ShapeDtypeStruct((N_NODES, C), f32),
  )(s1p, g1, dinv, b1p, W2p)

  s2p = _make_scatter()(src, dst, g2, zrows)

  y = pl.pallas_call(
      _tc4_body,
      out_shape=jax.ShapeDtypeStruct((N_NODES, 1), f32),
  )(s2p, g2, dinv, b2p, Wfcp, bfc.reshape(1, 1))
  return y
